# Initial kernel scaffold; baseline (speedup 1.0000x reference)
import jax
import jax.numpy as jnp
from jax.experimental import pallas as pl


def kernel(x, edge_index, lin0_W, lin0_b, W1_l1, W1_l2, lin1_W, lin1_b):
    raise NotImplementedError("write your pallas kernel here")


# speedup vs baseline: 2.9291x; 2.9291x over previous
import functools
import math

import jax
import jax.numpy as jnp
from jax import lax
from jax.experimental import pallas as pl
from jax.experimental.pallas import tpu as pltpu
from jax.experimental.pallas import tpu_sc as plsc

N = 10000
E = 320000
D = 128
ALPHA = 0.1
BETA1 = math.log(0.5 / 1.0 + 1.0)
BETA2 = math.log(0.5 / 2.0 + 1.0)

NC = 2
NS = 16
NW = NC * NS
CHUNK = 128
EDGES_PER_W = 10240
NCHUNK = EDGES_PER_W // CHUNK
E_PAD = EDGES_PER_W * NW
AGG_ROWS = 10112
STRIPE = AGG_ROWS // NS
DUMMY_ROW = N

ROW_BLK = 1000


def _sc_prop_body(h_hbm, src_hbm, dst_hbm, zero_hbm, out_hbm,
                  src_v, dst_v, rows_v, agg_sh, sem):
    c = lax.axis_index("c")
    s = lax.axis_index("s")
    wid = c * NS + s
    row0 = s * STRIPE

    pltpu.sync_copy(zero_hbm.at[pl.ds(row0, STRIPE)],
                    agg_sh.at[pl.ds(row0, STRIPE)])
    pltpu.sync_copy(src_hbm.at[wid], src_v)
    pltpu.sync_copy(dst_hbm.at[wid], dst_v)
    plsc.subcore_barrier()

    def body(j, carry):
        pltpu.async_copy(h_hbm.at[src_v.at[j]], rows_v, sem).wait()
        pltpu.sync_copy(rows_v, agg_sh.at[dst_v.at[j]], add=True)
        return carry

    lax.fori_loop(0, NCHUNK, body, 0)
    plsc.subcore_barrier()
    pltpu.sync_copy(agg_sh.at[pl.ds(row0, STRIPE)],
                    out_hbm.at[c, pl.ds(row0, STRIPE)])


_sc_prop = pl.kernel(
    _sc_prop_body,
    out_type=jax.ShapeDtypeStruct((NC, AGG_ROWS, D), jnp.float32),
    mesh=plsc.VectorSubcoreMesh(core_axis_name="c", subcore_axis_name="s"),
    scratch_types=[
        pltpu.VMEM((NCHUNK, CHUNK), jnp.int32),
        pltpu.VMEM((NCHUNK, CHUNK), jnp.int32),
        pltpu.VMEM((CHUNK, D), jnp.float32),
        pltpu.VMEM_SHARED((AGG_ROWS, D), jnp.float32),
        pltpu.SemaphoreType.DMA,
    ],
)


def _tc_in_body(x_ref, w_ref, b_ref, o_ref):
    y = lax.dot_general(x_ref[...], w_ref[...], (((1,), (1,)), ((), ())),
                        preferred_element_type=jnp.float32)
    o_ref[...] = jnp.maximum(y + b_ref[...], 0.0)


def _tc_in(x, w, b):
    return pl.pallas_call(
        _tc_in_body,
        grid=(N // ROW_BLK,),
        in_specs=[
            pl.BlockSpec((ROW_BLK, D), lambda i: (i, 0)),
            pl.BlockSpec((D, D), lambda i: (0, 0)),
            pl.BlockSpec((1, D), lambda i: (0, 0)),
        ],
        out_specs=pl.BlockSpec((ROW_BLK, D), lambda i: (i, 0)),
        out_shape=jax.ShapeDtypeStruct((N, D), jnp.float32),
    )(x, w, b.reshape(1, D))


def _tc_mid_body(p_ref, x0_ref, w1_ref, o_ref, *, beta):
    agg = p_ref[0] + p_ref[1]
    t = (1.0 - ALPHA) * agg + ALPHA * x0_ref[...]
    tw = jnp.dot(t, w1_ref[...], preferred_element_type=jnp.float32)
    o_ref[...] = jnp.maximum((1.0 - beta) * t + beta * tw, 0.0)


def _tc_mid(p, x0, w1, beta):
    return pl.pallas_call(
        functools.partial(_tc_mid_body, beta=beta),
        grid=(N // ROW_BLK,),
        in_specs=[
            pl.BlockSpec((NC, ROW_BLK, D), lambda i: (0, i, 0)),
            pl.BlockSpec((ROW_BLK, D), lambda i: (i, 0)),
            pl.BlockSpec((D, D), lambda i: (0, 0)),
        ],
        out_specs=pl.BlockSpec((ROW_BLK, D), lambda i: (i, 0)),
        out_shape=jax.ShapeDtypeStruct((N, D), jnp.float32),
    )(p, x0, w1)


def _tc_fin_body(p_ref, x0_ref, w1_ref, wo_ref, bo_ref, o_ref):
    agg = p_ref[0] + p_ref[1]
    t = (1.0 - ALPHA) * agg + ALPHA * x0_ref[...]
    tw = jnp.dot(t, w1_ref[...], preferred_element_type=jnp.float32)
    h = jnp.maximum((1.0 - BETA2) * t + BETA2 * tw, 0.0)
    o = lax.dot_general(h, wo_ref[...], (((1,), (1,)), ((), ())),
                        preferred_element_type=jnp.float32) + bo_ref[...]
    m = jnp.max(o, axis=-1, keepdims=True)
    lse = jnp.log(jnp.sum(jnp.exp(o - m), axis=-1, keepdims=True)) + m
    o_ref[...] = o - lse


def _tc_fin(p, x0, w1, wo, bo):
    return pl.pallas_call(
        _tc_fin_body,
        grid=(N // ROW_BLK,),
        in_specs=[
            pl.BlockSpec((NC, ROW_BLK, D), lambda i: (0, i, 0)),
            pl.BlockSpec((ROW_BLK, D), lambda i: (i, 0)),
            pl.BlockSpec((D, D), lambda i: (0, 0)),
            pl.BlockSpec((D, D), lambda i: (0, 0)),
            pl.BlockSpec((1, D), lambda i: (0, 0)),
        ],
        out_specs=pl.BlockSpec((ROW_BLK, D), lambda i: (i, 0)),
        out_shape=jax.ShapeDtypeStruct((N, D), jnp.float32),
    )(p, x0, w1, wo, bo.reshape(1, D))


def kernel(x, edge_index, lin0_W, lin0_b, W1_l1, W1_l2, lin1_W, lin1_b):
    src = edge_index[0]
    dst = edge_index[1]
    pad = E_PAD - E
    src_p = jnp.concatenate(
        [src, jnp.zeros((pad,), jnp.int32)]).reshape(NW, NCHUNK, CHUNK)
    dst_p = jnp.concatenate(
        [dst, jnp.full((pad,), DUMMY_ROW, jnp.int32)]).reshape(NW, NCHUNK, CHUNK)
    zero = jnp.zeros((AGG_ROWS, D), jnp.float32)

    x0 = _tc_in(x, lin0_W, lin0_b)
    p1 = _sc_prop(x0, src_p, dst_p, zero)
    h1 = _tc_mid(p1[:, :N, :], x0, W1_l1, BETA1)
    p2 = _sc_prop(h1, src_p, dst_p, zero)
    return _tc_fin(p2[:, :N, :], x0, W1_l2, lin1_W, lin1_b)


# speedup vs baseline: 3.1784x; 1.0851x over previous
import functools
import math

import jax
import jax.numpy as jnp
from jax import lax
from jax.experimental import pallas as pl
from jax.experimental.pallas import tpu as pltpu
from jax.experimental.pallas import tpu_sc as plsc

N = 10000
E = 320000
D = 128
ALPHA = 0.1
BETA1 = math.log(0.5 / 1.0 + 1.0)
BETA2 = math.log(0.5 / 2.0 + 1.0)

NC = 2
NS = 16
NW = NC * NS
CHUNK = 128
EDGES_PER_W = 10240
NCHUNK = EDGES_PER_W // CHUNK
E_PAD = EDGES_PER_W * NW
AGG_ROWS = 10112
STRIPE = AGG_ROWS // NS
DUMMY_ROW = N

ROW_BLK = 1000


def _sc_prop_body(h_hbm, src_hbm, dst_hbm, zero_hbm, out_hbm,
                  src_v, dst_v, rows_v, agg_sh, sem):
    c = lax.axis_index("c")
    s = lax.axis_index("s")
    wid = c * NS + s
    row0 = s * STRIPE

    pltpu.sync_copy(zero_hbm.at[pl.ds(row0, STRIPE)],
                    agg_sh.at[pl.ds(row0, STRIPE)])
    pltpu.sync_copy(src_hbm.at[wid], src_v)
    pltpu.sync_copy(dst_hbm.at[wid], dst_v)
    plsc.subcore_barrier()

    def body(j, carry):
        pltpu.async_copy(h_hbm.at[src_v.at[j]], rows_v, sem).wait()
        pltpu.sync_copy(rows_v, agg_sh.at[dst_v.at[j]], add=True)
        return carry

    lax.fori_loop(0, NCHUNK, body, 0)
    plsc.subcore_barrier()
    pltpu.sync_copy(agg_sh.at[pl.ds(row0, STRIPE)],
                    out_hbm.at[c, pl.ds(row0, STRIPE)])


_sc_prop = pl.kernel(
    _sc_prop_body,
    out_type=jax.ShapeDtypeStruct((NC, AGG_ROWS, D), jnp.float32),
    mesh=plsc.VectorSubcoreMesh(core_axis_name="c", subcore_axis_name="s"),
    scratch_types=[
        pltpu.VMEM((NCHUNK, CHUNK), jnp.int32),
        pltpu.VMEM((NCHUNK, CHUNK), jnp.int32),
        pltpu.VMEM((CHUNK, D), jnp.float32),
        pltpu.VMEM_SHARED((AGG_ROWS, D), jnp.float32),
        pltpu.SemaphoreType.DMA,
    ],
)


def _tc_in_body(x_ref, w_ref, b_ref, o_ref):
    y = lax.dot_general(x_ref[...], w_ref[...], (((1,), (1,)), ((), ())),
                        preferred_element_type=jnp.float32)
    o_ref[...] = jnp.maximum(y + b_ref[...], 0.0)


def _tc_in(x, w, b):
    return pl.pallas_call(
        _tc_in_body,
        grid=(N // ROW_BLK,),
        in_specs=[
            pl.BlockSpec((ROW_BLK, D), lambda i: (i, 0)),
            pl.BlockSpec((D, D), lambda i: (0, 0)),
            pl.BlockSpec((1, D), lambda i: (0, 0)),
        ],
        out_specs=pl.BlockSpec((ROW_BLK, D), lambda i: (i, 0)),
        out_shape=jax.ShapeDtypeStruct((N, D), jnp.float32),
    )(x, w, b.reshape(1, D))


def _tc_mid_body(p_ref, x0_ref, w1_ref, o_ref, *, beta):
    agg = p_ref[0] + p_ref[1]
    t = (1.0 - ALPHA) * agg + ALPHA * x0_ref[...]
    tw = jnp.dot(t, w1_ref[...], preferred_element_type=jnp.float32)
    o_ref[...] = jnp.maximum((1.0 - beta) * t + beta * tw, 0.0)


def _tc_mid(p, x0, w1, beta):
    return pl.pallas_call(
        functools.partial(_tc_mid_body, beta=beta),
        grid=(N // ROW_BLK,),
        in_specs=[
            pl.BlockSpec((NC, ROW_BLK, D), lambda i: (0, i, 0)),
            pl.BlockSpec((ROW_BLK, D), lambda i: (i, 0)),
            pl.BlockSpec((D, D), lambda i: (0, 0)),
        ],
        out_specs=pl.BlockSpec((ROW_BLK, D), lambda i: (i, 0)),
        out_shape=jax.ShapeDtypeStruct((N, D), jnp.float32),
    )(p, x0, w1)


def _tc_fin_body(p_ref, x0_ref, w1_ref, wo_ref, bo_ref, o_ref):
    agg = p_ref[0] + p_ref[1]
    t = (1.0 - ALPHA) * agg + ALPHA * x0_ref[...]
    tw = jnp.dot(t, w1_ref[...], preferred_element_type=jnp.float32)
    h = jnp.maximum((1.0 - BETA2) * t + BETA2 * tw, 0.0)
    o = lax.dot_general(h, wo_ref[...], (((1,), (1,)), ((), ())),
                        preferred_element_type=jnp.float32) + bo_ref[...]
    m = jnp.max(o, axis=-1, keepdims=True)
    lse = jnp.log(jnp.sum(jnp.exp(o - m), axis=-1, keepdims=True)) + m
    o_ref[...] = o - lse


def _tc_fin(p, x0, w1, wo, bo):
    return pl.pallas_call(
        _tc_fin_body,
        grid=(N // ROW_BLK,),
        in_specs=[
            pl.BlockSpec((NC, ROW_BLK, D), lambda i: (0, i, 0)),
            pl.BlockSpec((ROW_BLK, D), lambda i: (i, 0)),
            pl.BlockSpec((D, D), lambda i: (0, 0)),
            pl.BlockSpec((D, D), lambda i: (0, 0)),
            pl.BlockSpec((1, D), lambda i: (0, 0)),
        ],
        out_specs=pl.BlockSpec((ROW_BLK, D), lambda i: (i, 0)),
        out_shape=jax.ShapeDtypeStruct((N, D), jnp.float32),
    )(p, x0, w1, wo, bo.reshape(1, D))


def kernel(x, edge_index, lin0_W, lin0_b, W1_l1, W1_l2, lin1_W, lin1_b):
    src = edge_index[0]
    dst = edge_index[1]
    pad = E_PAD - E
    dummy = DUMMY_ROW + (jnp.arange(pad, dtype=jnp.int32) % (AGG_ROWS - N))
    src_p = jnp.concatenate(
        [src, jnp.zeros((pad,), jnp.int32)]
    ).reshape(NCHUNK, NW, CHUNK).swapaxes(0, 1)
    dst_p = jnp.concatenate(
        [dst, dummy]
    ).reshape(NCHUNK, NW, CHUNK).swapaxes(0, 1)
    zero = jnp.zeros((AGG_ROWS, D), jnp.float32)

    x0 = _tc_in(x, lin0_W, lin0_b)
    p1 = _sc_prop(x0, src_p, dst_p, zero)
    h1 = _tc_mid(p1[:, :N, :], x0, W1_l1, BETA1)
    p2 = _sc_prop(h1, src_p, dst_p, zero)
    return _tc_fin(p2[:, :N, :], x0, W1_l2, lin1_W, lin1_b)
